# baseline (speedup 1.0000x reference)
import functools

import jax
import jax.numpy as jnp
from jax import lax
from jax.experimental import pallas as pl
from jax.experimental.pallas import tpu as pltpu
from jax.experimental.pallas import tpu_sc as plsc

N = 10000
E = 320000
D = 128
NW = 32
K = 80
CH_C0 = 96
CH_C1 = 156
E_C0 = 16 * CH_C0 * K
E_PAD = 16 * (CH_C0 + CH_C1) * K
N_ACC = 10240
ROWS_PER_SUB = N_ACC // 16


def _tc_pre_body(x_ref, w1_ref, w2_ref, b_ref, h1_ref, h2_ref):
    x = x_ref[...]
    h1_ref[...] = jnp.dot(x, w1_ref[...], preferred_element_type=jnp.float32) + b_ref[...]
    h2_ref[...] = jnp.dot(x, w2_ref[...], preferred_element_type=jnp.float32)


def _tc_post_body(h1_ref, acc_ref, accw_ref, out_ref):
    s = acc_ref[0, :N, :] + acc_ref[1, :N, :]
    ws = (accw_ref[0, :N] + accw_ref[1, :N])[:, None]
    agg = s / jnp.maximum(ws, 1e-6)
    o = jnp.maximum(h1_ref[...] + agg, 0.0)
    nrm = jnp.sqrt(jnp.sum(o * o, axis=1, keepdims=True))
    out_ref[...] = o / jnp.maximum(nrm, 1e-12)


def _bcast_lane(v16, l):
    idx = jnp.full((16, 1), l, jnp.int32)
    dn = lax.GatherDimensionNumbers(
        offset_dims=(), collapsed_slice_dims=(0,), start_index_map=(0,))
    return lax.gather(v16, idx, dn, (1,),
                      mode=lax.GatherScatterMode.PROMISE_IN_BOUNDS)


def _sc_agg_body(h2_hbm, src_hbm, dst_hbm, w_hbm, zeros_hbm, out_hbm, outw_hbm,
                 rows0, rows1, rows2, dbuf0, dbuf1, dbuf2,
                 wbuf0, wbuf1, wbuf2,
                 fsrc0, fsrc1, fdst0, fdst1, fw0, fw1, wz_v,
                 acc_sh, accw_sh,
                 gsem0, gsem1, gsem2, ssem0, ssem1, ssem2,
                 wsem0, wsem1, wsem2, isem0, isem1):
    c = lax.axis_index("c")
    s = lax.axis_index("s")
    rows = (rows0, rows1, rows2)
    dbuf = (dbuf0, dbuf1, dbuf2)
    wbuf = (wbuf0, wbuf1, wbuf2)
    fsrc = (fsrc0, fsrc1)
    fdst = (fdst0, fdst1)
    fw = (fw0, fw1)
    gsem = (gsem0, gsem1, gsem2)
    ssem = (ssem0, ssem1, ssem2)
    wsem = (wsem0, wsem1, wsem2)
    isem = (isem0, isem1)

    ch_local = jnp.where(c == 0, CH_C0, CH_C1)
    base_e = jnp.where(c == 0, s * (CH_C0 * K), E_C0 + s * (CH_C1 * K))

    def fetch_start(i, f):
        pltpu.async_copy(src_hbm.at[pl.ds(base_e + i * K, K)], fsrc[f], isem[f])
        pltpu.async_copy(dst_hbm.at[pl.ds(base_e + i * K, K)], fdst[f], isem[f])
        pltpu.async_copy(w_hbm.at[pl.ds(base_e + i * K, K)], fw[f], isem[f])

    def fetch_wait(f):
        pltpu.make_async_copy(src_hbm.at[pl.ds(base_e, K)], fsrc[f],
                              isem[f]).wait()
        pltpu.make_async_copy(dst_hbm.at[pl.ds(base_e, K)], fdst[f],
                              isem[f]).wait()
        pltpu.make_async_copy(w_hbm.at[pl.ds(base_e, K)], fw[f],
                              isem[f]).wait()

    def gather_start(f, u):
        pltpu.async_copy(h2_hbm.at[fsrc[f]], rows[u], gsem[u])

    def gather_wait(u):
        pltpu.make_async_copy(h2_hbm.at[fsrc[0]], rows[u], gsem[u]).wait()

    def scatter_start(u):
        pltpu.async_copy(rows[u], acc_sh.at[dbuf[u]], ssem[u], add=True)
        pltpu.async_copy(wbuf[u], accw_sh.at[dbuf[u]], wsem[u], add=True)

    def scatter_wait(u):
        pltpu.make_async_copy(rows[u], acc_sh.at[dbuf[u]], ssem[u]).wait()
        pltpu.make_async_copy(wbuf[u], accw_sh.at[dbuf[u]], wsem[u]).wait()

    def compute(f, u):
        for g in range(K // 16):
            dbuf[u][pl.ds(g * 16, 16)] = fdst[f][pl.ds(g * 16, 16)]
            wbuf[u][pl.ds(g * 16, 16)] = fw[f][pl.ds(g * 16, 16)]

        def group(g, carry):
            w16 = wbuf[u][pl.ds(g * 16, 16)]
            for l in range(16):
                wb = _bcast_lane(w16, l)
                row = g * 16 + l
                for j in range(D // 16):
                    rows[u][row, pl.ds(j * 16, 16)] = (
                        rows[u][row, pl.ds(j * 16, 16)] * wb)
            return carry
        lax.fori_loop(0, K // 16, group, 0)

    fetch_start(0, 0)
    fetch_start(1, 1)

    pltpu.sync_copy(zeros_hbm, acc_sh.at[pl.ds(s * ROWS_PER_SUB, ROWS_PER_SUB)])
    for j in range(8):
        wz_v[pl.ds(j * 16, 16)] = jnp.zeros((16,), jnp.float32)
    for t in range(ROWS_PER_SUB // 128):
        pltpu.sync_copy(wz_v, accw_sh.at[pl.ds(s * ROWS_PER_SUB + t * 128, 128)])
    plsc.subcore_barrier()

    fetch_wait(0)
    gather_start(0, 0)

    def six(t, carry):
        for k in range(6):
            i = 6 * t + k
            u = k % 3
            f = k % 2
            gather_wait(u)
            if k >= 2:
                scatter_wait((u + 1) % 3)
            else:
                @pl.when(t >= 1)
                def _():
                    scatter_wait((u + 1) % 3)

            @pl.when(i + 1 < ch_local)
            def _():
                fetch_wait(1 - f)
                gather_start(1 - f, (u + 1) % 3)

            compute(f, u)
            scatter_start(u)

            @pl.when(i + 2 < ch_local)
            def _():
                fetch_start(i + 2, f)
        return carry
    lax.fori_loop(0, ch_local // 6, six, 0)

    scatter_wait(1)
    scatter_wait(2)

    plsc.subcore_barrier()
    pltpu.sync_copy(acc_sh.at[pl.ds(s * ROWS_PER_SUB, ROWS_PER_SUB)],
                    out_hbm.at[c, pl.ds(s * ROWS_PER_SUB, ROWS_PER_SUB)])
    for t in range(ROWS_PER_SUB // 128):
        pltpu.sync_copy(accw_sh.at[pl.ds(s * ROWS_PER_SUB + t * 128, 128)],
                        wz_v)
        pltpu.sync_copy(wz_v, outw_hbm.at[pl.ds(
            c * N_ACC + s * ROWS_PER_SUB + t * 128, 128)])


_BR = 1000


def _tc_pre(x, w1, w2, b):
    return pl.pallas_call(
        _tc_pre_body,
        grid=(N // _BR,),
        in_specs=[
            pl.BlockSpec((_BR, D), lambda i: (i, 0)),
            pl.BlockSpec((D, D), lambda i: (0, 0)),
            pl.BlockSpec((D, D), lambda i: (0, 0)),
            pl.BlockSpec((1, D), lambda i: (0, 0)),
        ],
        out_specs=[
            pl.BlockSpec((_BR, D), lambda i: (i, 0)),
            pl.BlockSpec((_BR, D), lambda i: (i, 0)),
        ],
        out_shape=[
            jax.ShapeDtypeStruct((N, D), jnp.float32),
            jax.ShapeDtypeStruct((N, D), jnp.float32),
        ],
    )(x, w1, w2, b)


_sc_agg = functools.partial(
    pl.kernel,
    out_type=[
        jax.ShapeDtypeStruct((2, N_ACC, D), jnp.float32),
        jax.ShapeDtypeStruct((2 * N_ACC,), jnp.float32),
    ],
    mesh=plsc.VectorSubcoreMesh(core_axis_name="c", subcore_axis_name="s"),
    scratch_types=[
        pltpu.VMEM((K, D), jnp.float32),
        pltpu.VMEM((K, D), jnp.float32),
        pltpu.VMEM((K, D), jnp.float32),
        pltpu.VMEM((K,), jnp.int32),
        pltpu.VMEM((K,), jnp.int32),
        pltpu.VMEM((K,), jnp.int32),
        pltpu.VMEM((K,), jnp.float32),
        pltpu.VMEM((K,), jnp.float32),
        pltpu.VMEM((K,), jnp.float32),
        pltpu.VMEM((K,), jnp.int32),
        pltpu.VMEM((K,), jnp.int32),
        pltpu.VMEM((K,), jnp.int32),
        pltpu.VMEM((K,), jnp.int32),
        pltpu.VMEM((K,), jnp.float32),
        pltpu.VMEM((K,), jnp.float32),
        pltpu.VMEM((128,), jnp.float32),
        pltpu.VMEM_SHARED((N_ACC, D), jnp.float32),
        pltpu.VMEM_SHARED((N_ACC,), jnp.float32),
        pltpu.SemaphoreType.DMA,
        pltpu.SemaphoreType.DMA,
        pltpu.SemaphoreType.DMA,
        pltpu.SemaphoreType.DMA,
        pltpu.SemaphoreType.DMA,
        pltpu.SemaphoreType.DMA,
        pltpu.SemaphoreType.DMA,
        pltpu.SemaphoreType.DMA,
        pltpu.SemaphoreType.DMA,
        pltpu.SemaphoreType.DMA,
        pltpu.SemaphoreType.DMA,
    ],
)(_sc_agg_body)


def _tc_post(h1, acc, accw):
    return pl.pallas_call(
        _tc_post_body,
        out_shape=jax.ShapeDtypeStruct((N, D), jnp.float32),
    )(h1, acc, accw)


def kernel(x, edge_index, edge_weight, kernel_1, kernel_2, bias):
    src = edge_index[0].astype(jnp.int32)
    dst = edge_index[1].astype(jnp.int32)
    pad = E_PAD - E
    src = jnp.concatenate([src, jnp.zeros((pad,), jnp.int32)])
    dst = jnp.concatenate([dst, jnp.zeros((pad,), jnp.int32)])
    w = jnp.concatenate([edge_weight.astype(jnp.float32),
                         jnp.zeros((pad,), jnp.float32)])
    zeros = jnp.zeros((ROWS_PER_SUB, D), jnp.float32)
    h1, h2 = _tc_pre(x, kernel_1, kernel_2, bias.reshape(1, D))
    acc, accw_flat = _sc_agg(h2, src, dst, w, zeros)
    return _tc_post(h1, acc, accw_flat.reshape(2, N_ACC))


# speedup vs baseline: 1.1474x; 1.1474x over previous
import functools

import jax
import jax.numpy as jnp
from jax import lax
from jax.experimental import pallas as pl
from jax.experimental.pallas import tpu as pltpu
from jax.experimental.pallas import tpu_sc as plsc

N = 10000
E = 320000
D = 128
NW = 32
K = 80
CH_C0 = 156
CH_C1 = 96
E_C0 = 16 * CH_C0 * K
E_PAD = 16 * (CH_C0 + CH_C1) * K
N_ACC = 10240
ROWS_PER_SUB = N_ACC // 16


def _tc_pre_body(x_ref, w1_ref, w2_ref, b_ref, h1_ref, h2_ref):
    x = x_ref[...]
    h1_ref[...] = jnp.dot(x, w1_ref[...], preferred_element_type=jnp.float32) + b_ref[...]
    h2_ref[...] = jnp.dot(x, w2_ref[...], preferred_element_type=jnp.float32)


def _tc_post_body(h1_ref, acc_ref, accw_ref, out_ref):
    s = acc_ref[0, :N, :] + acc_ref[1, :N, :]
    ws = (accw_ref[0, :N] + accw_ref[1, :N])[:, None]
    agg = s / jnp.maximum(ws, 1e-6)
    o = jnp.maximum(h1_ref[...] + agg, 0.0)
    nrm = jnp.sqrt(jnp.sum(o * o, axis=1, keepdims=True))
    out_ref[...] = o / jnp.maximum(nrm, 1e-12)


def _bcast_lane(v16, l):
    idx = jnp.full((16, 1), l, jnp.int32)
    dn = lax.GatherDimensionNumbers(
        offset_dims=(), collapsed_slice_dims=(0,), start_index_map=(0,))
    return lax.gather(v16, idx, dn, (1,),
                      mode=lax.GatherScatterMode.PROMISE_IN_BOUNDS)


def _sc_agg_body(h2_hbm, src_hbm, dst_hbm, w_hbm, zeros_hbm, out_hbm, outw_hbm,
                 rows0, rows1, rows2, dbuf0, dbuf1, dbuf2,
                 wbuf0, wbuf1, wbuf2,
                 fsrc0, fsrc1, fdst0, fdst1, fw0, fw1, wz_v,
                 acc_sh, accw_sh,
                 gsem0, gsem1, gsem2, ssem0, ssem1, ssem2,
                 wsem0, wsem1, wsem2, isem0, isem1):
    c = lax.axis_index("c")
    s = lax.axis_index("s")
    rows = (rows0, rows1, rows2)
    dbuf = (dbuf0, dbuf1, dbuf2)
    wbuf = (wbuf0, wbuf1, wbuf2)
    fsrc = (fsrc0, fsrc1)
    fdst = (fdst0, fdst1)
    fw = (fw0, fw1)
    gsem = (gsem0, gsem1, gsem2)
    ssem = (ssem0, ssem1, ssem2)
    wsem = (wsem0, wsem1, wsem2)
    isem = (isem0, isem1)

    ch_local = jnp.where(c == 0, CH_C0, CH_C1)
    base_e = jnp.where(c == 0, s * (CH_C0 * K), E_C0 + s * (CH_C1 * K))

    def fetch_start(i, f):
        pltpu.async_copy(src_hbm.at[pl.ds(base_e + i * K, K)], fsrc[f], isem[f])
        pltpu.async_copy(dst_hbm.at[pl.ds(base_e + i * K, K)], fdst[f], isem[f])
        pltpu.async_copy(w_hbm.at[pl.ds(base_e + i * K, K)], fw[f], isem[f])

    def fetch_wait(f):
        pltpu.make_async_copy(src_hbm.at[pl.ds(base_e, K)], fsrc[f],
                              isem[f]).wait()
        pltpu.make_async_copy(dst_hbm.at[pl.ds(base_e, K)], fdst[f],
                              isem[f]).wait()
        pltpu.make_async_copy(w_hbm.at[pl.ds(base_e, K)], fw[f],
                              isem[f]).wait()

    def gather_start(f, u):
        pltpu.async_copy(h2_hbm.at[fsrc[f]], rows[u], gsem[u])

    def gather_wait(u):
        pltpu.make_async_copy(h2_hbm.at[fsrc[0]], rows[u], gsem[u]).wait()

    def scatter_start(u):
        pltpu.async_copy(rows[u], acc_sh.at[dbuf[u]], ssem[u], add=True)
        pltpu.async_copy(wbuf[u], accw_sh.at[dbuf[u]], wsem[u], add=True)

    def scatter_wait(u):
        pltpu.make_async_copy(rows[u], acc_sh.at[dbuf[u]], ssem[u]).wait()
        pltpu.make_async_copy(wbuf[u], accw_sh.at[dbuf[u]], wsem[u]).wait()

    def compute(f, u):
        for g in range(K // 16):
            dbuf[u][pl.ds(g * 16, 16)] = fdst[f][pl.ds(g * 16, 16)]
            wbuf[u][pl.ds(g * 16, 16)] = fw[f][pl.ds(g * 16, 16)]

        def group(g, carry):
            w16 = wbuf[u][pl.ds(g * 16, 16)]
            for l in range(16):
                wb = _bcast_lane(w16, l)
                row = g * 16 + l
                for j in range(D // 16):
                    rows[u][row, pl.ds(j * 16, 16)] = (
                        rows[u][row, pl.ds(j * 16, 16)] * wb)
            return carry
        lax.fori_loop(0, K // 16, group, 0)

    fetch_start(0, 0)
    fetch_start(1, 1)

    pltpu.sync_copy(zeros_hbm, acc_sh.at[pl.ds(s * ROWS_PER_SUB, ROWS_PER_SUB)])
    for j in range(8):
        wz_v[pl.ds(j * 16, 16)] = jnp.zeros((16,), jnp.float32)
    for t in range(ROWS_PER_SUB // 128):
        pltpu.sync_copy(wz_v, accw_sh.at[pl.ds(s * ROWS_PER_SUB + t * 128, 128)])
    plsc.subcore_barrier()

    fetch_wait(0)
    gather_start(0, 0)

    def six(t, carry):
        for k in range(6):
            i = 6 * t + k
            u = k % 3
            f = k % 2
            gather_wait(u)
            if k >= 2:
                scatter_wait((u + 1) % 3)
            else:
                @pl.when(t >= 1)
                def _():
                    scatter_wait((u + 1) % 3)

            @pl.when(i + 1 < ch_local)
            def _():
                fetch_wait(1 - f)
                gather_start(1 - f, (u + 1) % 3)

            compute(f, u)
            scatter_start(u)

            @pl.when(i + 2 < ch_local)
            def _():
                fetch_start(i + 2, f)
        return carry
    lax.fori_loop(0, ch_local // 6, six, 0)

    scatter_wait(1)
    scatter_wait(2)

    plsc.subcore_barrier()
    pltpu.sync_copy(acc_sh.at[pl.ds(s * ROWS_PER_SUB, ROWS_PER_SUB)],
                    out_hbm.at[c, pl.ds(s * ROWS_PER_SUB, ROWS_PER_SUB)])
    for t in range(ROWS_PER_SUB // 128):
        pltpu.sync_copy(accw_sh.at[pl.ds(s * ROWS_PER_SUB + t * 128, 128)],
                        wz_v)
        pltpu.sync_copy(wz_v, outw_hbm.at[pl.ds(
            c * N_ACC + s * ROWS_PER_SUB + t * 128, 128)])


_BR = 1000


def _tc_pre(x, w1, w2, b):
    return pl.pallas_call(
        _tc_pre_body,
        grid=(N // _BR,),
        in_specs=[
            pl.BlockSpec((_BR, D), lambda i: (i, 0)),
            pl.BlockSpec((D, D), lambda i: (0, 0)),
            pl.BlockSpec((D, D), lambda i: (0, 0)),
            pl.BlockSpec((1, D), lambda i: (0, 0)),
        ],
        out_specs=[
            pl.BlockSpec((_BR, D), lambda i: (i, 0)),
            pl.BlockSpec((_BR, D), lambda i: (i, 0)),
        ],
        out_shape=[
            jax.ShapeDtypeStruct((N, D), jnp.float32),
            jax.ShapeDtypeStruct((N, D), jnp.float32),
        ],
    )(x, w1, w2, b)


_sc_agg = functools.partial(
    pl.kernel,
    out_type=[
        jax.ShapeDtypeStruct((2, N_ACC, D), jnp.float32),
        jax.ShapeDtypeStruct((2 * N_ACC,), jnp.float32),
    ],
    mesh=plsc.VectorSubcoreMesh(core_axis_name="c", subcore_axis_name="s"),
    scratch_types=[
        pltpu.VMEM((K, D), jnp.float32),
        pltpu.VMEM((K, D), jnp.float32),
        pltpu.VMEM((K, D), jnp.float32),
        pltpu.VMEM((K,), jnp.int32),
        pltpu.VMEM((K,), jnp.int32),
        pltpu.VMEM((K,), jnp.int32),
        pltpu.VMEM((K,), jnp.float32),
        pltpu.VMEM((K,), jnp.float32),
        pltpu.VMEM((K,), jnp.float32),
        pltpu.VMEM((K,), jnp.int32),
        pltpu.VMEM((K,), jnp.int32),
        pltpu.VMEM((K,), jnp.int32),
        pltpu.VMEM((K,), jnp.int32),
        pltpu.VMEM((K,), jnp.float32),
        pltpu.VMEM((K,), jnp.float32),
        pltpu.VMEM((128,), jnp.float32),
        pltpu.VMEM_SHARED((N_ACC, D), jnp.float32),
        pltpu.VMEM_SHARED((N_ACC,), jnp.float32),
        pltpu.SemaphoreType.DMA,
        pltpu.SemaphoreType.DMA,
        pltpu.SemaphoreType.DMA,
        pltpu.SemaphoreType.DMA,
        pltpu.SemaphoreType.DMA,
        pltpu.SemaphoreType.DMA,
        pltpu.SemaphoreType.DMA,
        pltpu.SemaphoreType.DMA,
        pltpu.SemaphoreType.DMA,
        pltpu.SemaphoreType.DMA,
        pltpu.SemaphoreType.DMA,
    ],
)(_sc_agg_body)


def _tc_post(h1, acc, accw):
    return pl.pallas_call(
        _tc_post_body,
        out_shape=jax.ShapeDtypeStruct((N, D), jnp.float32),
    )(h1, acc, accw)


def kernel(x, edge_index, edge_weight, kernel_1, kernel_2, bias):
    src = edge_index[0].astype(jnp.int32)
    dst = edge_index[1].astype(jnp.int32)
    pad = E_PAD - E
    src = jnp.concatenate([src, jnp.zeros((pad,), jnp.int32)])
    dst = jnp.concatenate([dst, jnp.zeros((pad,), jnp.int32)])
    w = jnp.concatenate([edge_weight.astype(jnp.float32),
                         jnp.zeros((pad,), jnp.float32)])
    zeros = jnp.zeros((ROWS_PER_SUB, D), jnp.float32)
    h1, h2 = _tc_pre(x, kernel_1, kernel_2, bias.reshape(1, D))
    acc, accw_flat = _sc_agg(h2, src, dst, w, zeros)
    return _tc_post(h1, acc, accw_flat.reshape(2, N_ACC))


# speedup vs baseline: 1.1960x; 1.0423x over previous
import functools

import jax
import jax.numpy as jnp
from jax import lax
from jax.experimental import pallas as pl
from jax.experimental.pallas import tpu as pltpu
from jax.experimental.pallas import tpu_sc as plsc

N = 10000
E = 320000
D = 128
NW = 32
K = 80
CH_C0 = 162
CH_C1 = 90
E_C0 = 16 * CH_C0 * K
E_PAD = 16 * (CH_C0 + CH_C1) * K
N_ACC = 10240
ROWS_PER_SUB = N_ACC // 16


def _tc_pre_body(x_ref, w1_ref, w2_ref, b_ref, h1_ref, h2_ref):
    x = x_ref[...]
    h1_ref[...] = jnp.dot(x, w1_ref[...], preferred_element_type=jnp.float32) + b_ref[...]
    h2_ref[...] = jnp.dot(x, w2_ref[...], preferred_element_type=jnp.float32)


def _tc_post_body(h1_ref, acc_ref, accw_ref, out_ref):
    s = acc_ref[0, :N, :] + acc_ref[1, :N, :]
    ws = (accw_ref[0, :N] + accw_ref[1, :N])[:, None]
    agg = s / jnp.maximum(ws, 1e-6)
    o = jnp.maximum(h1_ref[...] + agg, 0.0)
    nrm = jnp.sqrt(jnp.sum(o * o, axis=1, keepdims=True))
    out_ref[...] = o / jnp.maximum(nrm, 1e-12)


def _bcast_lane(v16, l):
    idx = jnp.full((16, 1), l, jnp.int32)
    dn = lax.GatherDimensionNumbers(
        offset_dims=(), collapsed_slice_dims=(0,), start_index_map=(0,))
    return lax.gather(v16, idx, dn, (1,),
                      mode=lax.GatherScatterMode.PROMISE_IN_BOUNDS)


def _sc_agg_body(h2_hbm, src_hbm, dst_hbm, w_hbm, zeros_hbm, out_hbm, outw_hbm,
                 rows0, rows1, rows2, dbuf0, dbuf1, dbuf2,
                 wbuf0, wbuf1, wbuf2,
                 fsrc0, fsrc1, fdst0, fdst1, fw0, fw1, wz_v,
                 acc_sh, accw_sh,
                 gsem0, gsem1, gsem2, ssem0, ssem1, ssem2,
                 wsem0, wsem1, wsem2, isem0, isem1):
    c = lax.axis_index("c")
    s = lax.axis_index("s")
    rows = (rows0, rows1, rows2)
    dbuf = (dbuf0, dbuf1, dbuf2)
    wbuf = (wbuf0, wbuf1, wbuf2)
    fsrc = (fsrc0, fsrc1)
    fdst = (fdst0, fdst1)
    fw = (fw0, fw1)
    gsem = (gsem0, gsem1, gsem2)
    ssem = (ssem0, ssem1, ssem2)
    wsem = (wsem0, wsem1, wsem2)
    isem = (isem0, isem1)

    ch_local = jnp.where(c == 0, CH_C0, CH_C1)
    base_e = jnp.where(c == 0, s * (CH_C0 * K), E_C0 + s * (CH_C1 * K))

    def fetch_start(i, f):
        pltpu.async_copy(src_hbm.at[pl.ds(base_e + i * K, K)], fsrc[f], isem[f])
        pltpu.async_copy(dst_hbm.at[pl.ds(base_e + i * K, K)], fdst[f], isem[f])
        pltpu.async_copy(w_hbm.at[pl.ds(base_e + i * K, K)], fw[f], isem[f])

    def fetch_wait(f):
        pltpu.make_async_copy(src_hbm.at[pl.ds(base_e, K)], fsrc[f],
                              isem[f]).wait()
        pltpu.make_async_copy(dst_hbm.at[pl.ds(base_e, K)], fdst[f],
                              isem[f]).wait()
        pltpu.make_async_copy(w_hbm.at[pl.ds(base_e, K)], fw[f],
                              isem[f]).wait()

    def gather_start(f, u):
        pltpu.async_copy(h2_hbm.at[fsrc[f]], rows[u], gsem[u])

    def gather_wait(u):
        pltpu.make_async_copy(h2_hbm.at[fsrc[0]], rows[u], gsem[u]).wait()

    def scatter_start(u):
        pltpu.async_copy(rows[u], acc_sh.at[dbuf[u]], ssem[u], add=True)
        pltpu.async_copy(wbuf[u], accw_sh.at[dbuf[u]], wsem[u], add=True)

    def scatter_wait(u):
        pltpu.make_async_copy(rows[u], acc_sh.at[dbuf[u]], ssem[u]).wait()
        pltpu.make_async_copy(wbuf[u], accw_sh.at[dbuf[u]], wsem[u]).wait()

    def compute(f, u):
        for g in range(K // 16):
            dbuf[u][pl.ds(g * 16, 16)] = fdst[f][pl.ds(g * 16, 16)]
            wbuf[u][pl.ds(g * 16, 16)] = fw[f][pl.ds(g * 16, 16)]

        def group(g, carry):
            w16 = wbuf[u][pl.ds(g * 16, 16)]
            for l in range(16):
                wb = _bcast_lane(w16, l)
                row = g * 16 + l
                for j in range(D // 16):
                    rows[u][row, pl.ds(j * 16, 16)] = (
                        rows[u][row, pl.ds(j * 16, 16)] * wb)
            return carry
        lax.fori_loop(0, K // 16, group, 0)

    fetch_start(0, 0)
    fetch_start(1, 1)

    pltpu.sync_copy(zeros_hbm, acc_sh.at[pl.ds(s * ROWS_PER_SUB, ROWS_PER_SUB)])
    for j in range(8):
        wz_v[pl.ds(j * 16, 16)] = jnp.zeros((16,), jnp.float32)
    for t in range(ROWS_PER_SUB // 128):
        pltpu.sync_copy(wz_v, accw_sh.at[pl.ds(s * ROWS_PER_SUB + t * 128, 128)])
    plsc.subcore_barrier()

    fetch_wait(0)
    gather_start(0, 0)

    def six(t, carry):
        for k in range(6):
            i = 6 * t + k
            u = k % 3
            f = k % 2
            gather_wait(u)
            if k >= 2:
                scatter_wait((u + 1) % 3)
            else:
                @pl.when(t >= 1)
                def _():
                    scatter_wait((u + 1) % 3)

            @pl.when(i + 1 < ch_local)
            def _():
                fetch_wait(1 - f)
                gather_start(1 - f, (u + 1) % 3)

            compute(f, u)
            scatter_start(u)

            @pl.when(i + 2 < ch_local)
            def _():
                fetch_start(i + 2, f)
        return carry
    lax.fori_loop(0, ch_local // 6, six, 0)

    scatter_wait(1)
    scatter_wait(2)

    plsc.subcore_barrier()
    pltpu.sync_copy(acc_sh.at[pl.ds(s * ROWS_PER_SUB, ROWS_PER_SUB)],
                    out_hbm.at[c, pl.ds(s * ROWS_PER_SUB, ROWS_PER_SUB)])
    for t in range(ROWS_PER_SUB // 128):
        pltpu.sync_copy(accw_sh.at[pl.ds(s * ROWS_PER_SUB + t * 128, 128)],
                        wz_v)
        pltpu.sync_copy(wz_v, outw_hbm.at[pl.ds(
            c * N_ACC + s * ROWS_PER_SUB + t * 128, 128)])


_BR = 1000


def _tc_pre(x, w1, w2, b):
    return pl.pallas_call(
        _tc_pre_body,
        grid=(N // _BR,),
        in_specs=[
            pl.BlockSpec((_BR, D), lambda i: (i, 0)),
            pl.BlockSpec((D, D), lambda i: (0, 0)),
            pl.BlockSpec((D, D), lambda i: (0, 0)),
            pl.BlockSpec((1, D), lambda i: (0, 0)),
        ],
        out_specs=[
            pl.BlockSpec((_BR, D), lambda i: (i, 0)),
            pl.BlockSpec((_BR, D), lambda i: (i, 0)),
        ],
        out_shape=[
            jax.ShapeDtypeStruct((N, D), jnp.float32),
            jax.ShapeDtypeStruct((N, D), jnp.float32),
        ],
    )(x, w1, w2, b)


_sc_agg = functools.partial(
    pl.kernel,
    out_type=[
        jax.ShapeDtypeStruct((2, N_ACC, D), jnp.float32),
        jax.ShapeDtypeStruct((2 * N_ACC,), jnp.float32),
    ],
    mesh=plsc.VectorSubcoreMesh(core_axis_name="c", subcore_axis_name="s"),
    scratch_types=[
        pltpu.VMEM((K, D), jnp.float32),
        pltpu.VMEM((K, D), jnp.float32),
        pltpu.VMEM((K, D), jnp.float32),
        pltpu.VMEM((K,), jnp.int32),
        pltpu.VMEM((K,), jnp.int32),
        pltpu.VMEM((K,), jnp.int32),
        pltpu.VMEM((K,), jnp.float32),
        pltpu.VMEM((K,), jnp.float32),
        pltpu.VMEM((K,), jnp.float32),
        pltpu.VMEM((K,), jnp.int32),
        pltpu.VMEM((K,), jnp.int32),
        pltpu.VMEM((K,), jnp.int32),
        pltpu.VMEM((K,), jnp.int32),
        pltpu.VMEM((K,), jnp.float32),
        pltpu.VMEM((K,), jnp.float32),
        pltpu.VMEM((128,), jnp.float32),
        pltpu.VMEM_SHARED((N_ACC, D), jnp.float32),
        pltpu.VMEM_SHARED((N_ACC,), jnp.float32),
        pltpu.SemaphoreType.DMA,
        pltpu.SemaphoreType.DMA,
        pltpu.SemaphoreType.DMA,
        pltpu.SemaphoreType.DMA,
        pltpu.SemaphoreType.DMA,
        pltpu.SemaphoreType.DMA,
        pltpu.SemaphoreType.DMA,
        pltpu.SemaphoreType.DMA,
        pltpu.SemaphoreType.DMA,
        pltpu.SemaphoreType.DMA,
        pltpu.SemaphoreType.DMA,
    ],
)(_sc_agg_body)


def _tc_post(h1, acc, accw):
    return pl.pallas_call(
        _tc_post_body,
        out_shape=jax.ShapeDtypeStruct((N, D), jnp.float32),
    )(h1, acc, accw)


def kernel(x, edge_index, edge_weight, kernel_1, kernel_2, bias):
    src = edge_index[0].astype(jnp.int32)
    dst = edge_index[1].astype(jnp.int32)
    pad = E_PAD - E
    src = jnp.concatenate([src, jnp.zeros((pad,), jnp.int32)])
    dst = jnp.concatenate([dst, jnp.zeros((pad,), jnp.int32)])
    w = jnp.concatenate([edge_weight.astype(jnp.float32),
                         jnp.zeros((pad,), jnp.float32)])
    zeros = jnp.zeros((ROWS_PER_SUB, D), jnp.float32)
    h1, h2 = _tc_pre(x, kernel_1, kernel_2, bias.reshape(1, D))
    acc, accw_flat = _sc_agg(h2, src, dst, w, zeros)
    return _tc_post(h1, acc, accw_flat.reshape(2, N_ACC))


# speedup vs baseline: 1.3551x; 1.1330x over previous
import functools

import jax
import jax.numpy as jnp
from jax import lax
from jax.experimental import pallas as pl
from jax.experimental.pallas import tpu as pltpu
from jax.experimental.pallas import tpu_sc as plsc

N = 10000
E = 320000
D = 128
NW = 32
K = 80
CH_C0 = 168
CH_C1 = 82
E_C0 = 16 * CH_C0 * K
N_ACC = 10240
ROWS_PER_SUB = N_ACC // 16


def _tc_pre_body(x_ref, w1_ref, w2_ref, b_ref, h1_ref, h2_ref):
    x = x_ref[...]
    h1_ref[...] = jnp.dot(x, w1_ref[...], preferred_element_type=jnp.float32) + b_ref[...]
    h2_ref[...] = jnp.dot(x, w2_ref[...], preferred_element_type=jnp.float32)


def _tc_post_body(h1_ref, acc_ref, accw_ref, out_ref):
    s = acc_ref[0, :N, :] + acc_ref[1, :N, :]
    ws = (accw_ref[0, :N] + accw_ref[1, :N])[:, None]
    agg = s / jnp.maximum(ws, 1e-6)
    o = jnp.maximum(h1_ref[...] + agg, 0.0)
    nrm = jnp.sqrt(jnp.sum(o * o, axis=1, keepdims=True))
    out_ref[...] = o / jnp.maximum(nrm, 1e-12)


def _bcast_lane(v16, l):
    idx = jnp.full((16, 1), l, jnp.int32)
    dn = lax.GatherDimensionNumbers(
        offset_dims=(), collapsed_slice_dims=(0,), start_index_map=(0,))
    return lax.gather(v16, idx, dn, (1,),
                      mode=lax.GatherScatterMode.PROMISE_IN_BOUNDS)


def _sc_agg_body(h2_hbm, ei_hbm, w_hbm, zeros_hbm, out_hbm, outw_hbm,
                 rows0, rows1, rows2, dbuf0, dbuf1, dbuf2,
                 wbuf0, wbuf1, wbuf2,
                 fsrc0, fsrc1, fdst0, fdst1, fw0, fw1, wz_v,
                 acc_sh, accw_sh,
                 gsem0, gsem1, gsem2, ssem0, ssem1, ssem2,
                 wsem0, wsem1, wsem2, isem0, isem1):
    c = lax.axis_index("c")
    s = lax.axis_index("s")
    rows = (rows0, rows1, rows2)
    dbuf = (dbuf0, dbuf1, dbuf2)
    wbuf = (wbuf0, wbuf1, wbuf2)
    fsrc = (fsrc0, fsrc1)
    fdst = (fdst0, fdst1)
    fw = (fw0, fw1)
    gsem = (gsem0, gsem1, gsem2)
    ssem = (ssem0, ssem1, ssem2)
    wsem = (wsem0, wsem1, wsem2)
    isem = (isem0, isem1)

    ch_local = jnp.where(c == 0, CH_C0, CH_C1)
    base_e = jnp.where(c == 0, s * (CH_C0 * K), E_C0 + s * (CH_C1 * K))

    def fetch_start(i, f):
        pltpu.async_copy(ei_hbm.at[pl.ds(base_e + i * K, K)], fsrc[f], isem[f])
        pltpu.async_copy(ei_hbm.at[pl.ds(E + base_e + i * K, K)], fdst[f],
                         isem[f])
        pltpu.async_copy(w_hbm.at[pl.ds(base_e + i * K, K)], fw[f], isem[f])

    def fetch_wait(f):
        pltpu.make_async_copy(ei_hbm.at[pl.ds(0, K)], fsrc[f], isem[f]).wait()
        pltpu.make_async_copy(ei_hbm.at[pl.ds(0, K)], fdst[f], isem[f]).wait()
        pltpu.make_async_copy(w_hbm.at[pl.ds(0, K)], fw[f], isem[f]).wait()

    def gather_start(f, u):
        pltpu.async_copy(h2_hbm.at[fsrc[f]], rows[u], gsem[u])

    def gather_wait(u):
        pltpu.make_async_copy(h2_hbm.at[fsrc[0]], rows[u], gsem[u]).wait()

    def scatter_start(u):
        pltpu.async_copy(rows[u], acc_sh.at[dbuf[u]], ssem[u], add=True)
        pltpu.async_copy(wbuf[u], accw_sh.at[dbuf[u]], wsem[u], add=True)

    def scatter_wait(u):
        pltpu.make_async_copy(rows[u], acc_sh.at[dbuf[u]], ssem[u]).wait()
        pltpu.make_async_copy(wbuf[u], accw_sh.at[dbuf[u]], wsem[u]).wait()

    def compute(f, u):
        for g in range(K // 16):
            dbuf[u][pl.ds(g * 16, 16)] = fdst[f][pl.ds(g * 16, 16)]
            wbuf[u][pl.ds(g * 16, 16)] = fw[f][pl.ds(g * 16, 16)]

        def group(g, carry):
            w16 = wbuf[u][pl.ds(g * 16, 16)]
            for l in range(16):
                wb = _bcast_lane(w16, l)
                row = g * 16 + l
                for j in range(D // 16):
                    rows[u][row, pl.ds(j * 16, 16)] = (
                        rows[u][row, pl.ds(j * 16, 16)] * wb)
            return carry
        lax.fori_loop(0, K // 16, group, 0)

    fetch_start(0, 0)
    fetch_start(1, 1)

    pltpu.sync_copy(zeros_hbm, acc_sh.at[pl.ds(s * ROWS_PER_SUB, ROWS_PER_SUB)])
    for j in range(8):
        wz_v[pl.ds(j * 16, 16)] = jnp.zeros((16,), jnp.float32)
    for t in range(ROWS_PER_SUB // 128):
        pltpu.sync_copy(wz_v, accw_sh.at[pl.ds(s * ROWS_PER_SUB + t * 128, 128)])
    plsc.subcore_barrier()

    fetch_wait(0)
    gather_start(0, 0)

    def six(t, carry):
        for k in range(6):
            i = 6 * t + k
            u = k % 3
            f = k % 2

            @pl.when(i < ch_local)
            def _():
                gather_wait(u)

            if k >= 2:
                @pl.when(i < ch_local)
                def _():
                    scatter_wait((u + 1) % 3)
            else:
                @pl.when(jnp.logical_and(t >= 1, i < ch_local))
                def _():
                    scatter_wait((u + 1) % 3)

            @pl.when(i + 1 < ch_local)
            def _():
                fetch_wait(1 - f)
                gather_start(1 - f, (u + 1) % 3)

            @pl.when(i < ch_local)
            def _():
                compute(f, u)
                scatter_start(u)

            @pl.when(i + 2 < ch_local)
            def _():
                fetch_start(i + 2, f)
        return carry
    lax.fori_loop(0, (ch_local + 5) // 6, six, 0)

    @pl.when(c == 0)
    def _():
        scatter_wait((CH_C0 - 2) % 3)
        scatter_wait((CH_C0 - 1) % 3)

    @pl.when(c == 1)
    def _():
        scatter_wait((CH_C1 - 2) % 3)
        scatter_wait((CH_C1 - 1) % 3)

    plsc.subcore_barrier()
    pltpu.sync_copy(acc_sh.at[pl.ds(s * ROWS_PER_SUB, ROWS_PER_SUB)],
                    out_hbm.at[c, pl.ds(s * ROWS_PER_SUB, ROWS_PER_SUB)])
    for t in range(ROWS_PER_SUB // 128):
        pltpu.sync_copy(accw_sh.at[pl.ds(s * ROWS_PER_SUB + t * 128, 128)],
                        wz_v)
        pltpu.sync_copy(wz_v, outw_hbm.at[pl.ds(
            c * N_ACC + s * ROWS_PER_SUB + t * 128, 128)])


_BR = 1000


def _tc_pre(x, w1, w2, b):
    return pl.pallas_call(
        _tc_pre_body,
        grid=(N // _BR,),
        in_specs=[
            pl.BlockSpec((_BR, D), lambda i: (i, 0)),
            pl.BlockSpec((D, D), lambda i: (0, 0)),
            pl.BlockSpec((D, D), lambda i: (0, 0)),
            pl.BlockSpec((1, D), lambda i: (0, 0)),
        ],
        out_specs=[
            pl.BlockSpec((_BR, D), lambda i: (i, 0)),
            pl.BlockSpec((_BR, D), lambda i: (i, 0)),
        ],
        out_shape=[
            jax.ShapeDtypeStruct((N, D), jnp.float32),
            jax.ShapeDtypeStruct((N, D), jnp.float32),
        ],
    )(x, w1, w2, b)


_sc_agg = functools.partial(
    pl.kernel,
    out_type=[
        jax.ShapeDtypeStruct((2, N_ACC, D), jnp.float32),
        jax.ShapeDtypeStruct((2 * N_ACC,), jnp.float32),
    ],
    mesh=plsc.VectorSubcoreMesh(core_axis_name="c", subcore_axis_name="s"),
    scratch_types=[
        pltpu.VMEM((K, D), jnp.float32),
        pltpu.VMEM((K, D), jnp.float32),
        pltpu.VMEM((K, D), jnp.float32),
        pltpu.VMEM((K,), jnp.int32),
        pltpu.VMEM((K,), jnp.int32),
        pltpu.VMEM((K,), jnp.int32),
        pltpu.VMEM((K,), jnp.float32),
        pltpu.VMEM((K,), jnp.float32),
        pltpu.VMEM((K,), jnp.float32),
        pltpu.VMEM((K,), jnp.int32),
        pltpu.VMEM((K,), jnp.int32),
        pltpu.VMEM((K,), jnp.int32),
        pltpu.VMEM((K,), jnp.int32),
        pltpu.VMEM((K,), jnp.float32),
        pltpu.VMEM((K,), jnp.float32),
        pltpu.VMEM((128,), jnp.float32),
        pltpu.VMEM_SHARED((N_ACC, D), jnp.float32),
        pltpu.VMEM_SHARED((N_ACC,), jnp.float32),
        pltpu.SemaphoreType.DMA,
        pltpu.SemaphoreType.DMA,
        pltpu.SemaphoreType.DMA,
        pltpu.SemaphoreType.DMA,
        pltpu.SemaphoreType.DMA,
        pltpu.SemaphoreType.DMA,
        pltpu.SemaphoreType.DMA,
        pltpu.SemaphoreType.DMA,
        pltpu.SemaphoreType.DMA,
        pltpu.SemaphoreType.DMA,
        pltpu.SemaphoreType.DMA,
    ],
)(_sc_agg_body)


def _tc_post(h1, acc, accw):
    return pl.pallas_call(
        _tc_post_body,
        out_shape=jax.ShapeDtypeStruct((N, D), jnp.float32),
    )(h1, acc, accw)


def kernel(x, edge_index, edge_weight, kernel_1, kernel_2, bias):
    ei = edge_index.astype(jnp.int32).reshape(2 * E)
    w = edge_weight.astype(jnp.float32)
    zeros = jnp.zeros((ROWS_PER_SUB, D), jnp.float32)
    h1, h2 = _tc_pre(x, kernel_1, kernel_2, bias.reshape(1, D))
    acc, accw_flat = _sc_agg(h2, ei, w, zeros)
    return _tc_post(h1, acc, accw_flat.reshape(2, N_ACC))


# speedup vs baseline: 1.6194x; 1.1951x over previous
import functools

import jax
import jax.numpy as jnp
from jax import lax
from jax.experimental import pallas as pl
from jax.experimental.pallas import tpu as pltpu
from jax.experimental.pallas import tpu_sc as plsc

N = 10000
E = 320000
D = 128
NW = 32
K = 80
CH_C0 = 130
CH_C1 = 120
E_C0 = 16 * CH_C0 * K
N_ACC = 10240
ROWS_PER_SUB = N_ACC // 16


def _tc_pre_body(x_ref, w1_ref, w2_ref, b_ref, h1_ref, h2_ref):
    x = x_ref[...]
    h1_ref[...] = jnp.dot(x, w1_ref[...], preferred_element_type=jnp.float32) + b_ref[...]
    h2_ref[...] = jnp.dot(x, w2_ref[...], preferred_element_type=jnp.float32)


def _tc_post_body(h1_ref, acc_ref, accw_ref, out_ref):
    s = acc_ref[0, :N, :] + acc_ref[1, :N, :]
    ws = (accw_ref[0, :N] + accw_ref[1, :N])[:, None]
    agg = s / jnp.maximum(ws, 1e-6)
    o = jnp.maximum(h1_ref[...] + agg, 0.0)
    nrm = jnp.sqrt(jnp.sum(o * o, axis=1, keepdims=True))
    out_ref[...] = o / jnp.maximum(nrm, 1e-12)


def _bcast_lane(v16, l):
    idx = jnp.full((16, 1), l, jnp.int32)
    dn = lax.GatherDimensionNumbers(
        offset_dims=(), collapsed_slice_dims=(0,), start_index_map=(0,))
    return lax.gather(v16, idx, dn, (1,),
                      mode=lax.GatherScatterMode.PROMISE_IN_BOUNDS)


def _sc_agg_body(h2_hbm, ei_hbm, w_hbm, zeros_hbm, out_hbm, outw_hbm,
                 rows0, rows1, rows2, dbuf0, dbuf1, dbuf2,
                 wbuf0, wbuf1, wbuf2,
                 fsrc0, fsrc1, fdst0, fdst1, fw0, fw1, wz_v,
                 acc_sh, accw_sh,
                 gsem0, gsem1, gsem2, ssem0, ssem1, ssem2,
                 wsem0, wsem1, wsem2, isem0, isem1):
    c = lax.axis_index("c")
    s = lax.axis_index("s")
    rows = (rows0, rows1, rows2)
    dbuf = (dbuf0, dbuf1, dbuf2)
    wbuf = (wbuf0, wbuf1, wbuf2)
    fsrc = (fsrc0, fsrc1)
    fdst = (fdst0, fdst1)
    fw = (fw0, fw1)
    gsem = (gsem0, gsem1, gsem2)
    ssem = (ssem0, ssem1, ssem2)
    wsem = (wsem0, wsem1, wsem2)
    isem = (isem0, isem1)

    ch_local = jnp.where(c == 0, CH_C0, CH_C1)
    base_e = jnp.where(c == 0, s * (CH_C0 * K), E_C0 + s * (CH_C1 * K))

    def fetch_start(i, f):
        pltpu.async_copy(ei_hbm.at[pl.ds(base_e + i * K, K)], fsrc[f], isem[f])
        pltpu.async_copy(ei_hbm.at[pl.ds(E + base_e + i * K, K)], fdst[f],
                         isem[f])
        pltpu.async_copy(w_hbm.at[pl.ds(base_e + i * K, K)], fw[f], isem[f])

    def fetch_wait(f):
        pltpu.make_async_copy(ei_hbm.at[pl.ds(0, K)], fsrc[f], isem[f]).wait()
        pltpu.make_async_copy(ei_hbm.at[pl.ds(0, K)], fdst[f], isem[f]).wait()
        pltpu.make_async_copy(w_hbm.at[pl.ds(0, K)], fw[f], isem[f]).wait()

    def gather_start(f, u):
        pltpu.async_copy(h2_hbm.at[fsrc[f]], rows[u], gsem[u])

    def gather_wait(u):
        pltpu.make_async_copy(h2_hbm.at[fsrc[0]], rows[u], gsem[u]).wait()

    def scatter_start(u):
        pltpu.async_copy(rows[u], acc_sh.at[dbuf[u]], ssem[u], add=True)
        pltpu.async_copy(wbuf[u], accw_sh.at[dbuf[u]], wsem[u], add=True)

    def scatter_wait(u):
        pltpu.make_async_copy(rows[u], acc_sh.at[dbuf[u]], ssem[u]).wait()
        pltpu.make_async_copy(wbuf[u], accw_sh.at[dbuf[u]], wsem[u]).wait()

    def compute(f, u):
        for g in range(K // 16):
            dbuf[u][pl.ds(g * 16, 16)] = fdst[f][pl.ds(g * 16, 16)]
            wbuf[u][pl.ds(g * 16, 16)] = fw[f][pl.ds(g * 16, 16)]

        def group(g, carry):
            w16 = wbuf[u][pl.ds(g * 16, 16)]
            for l in range(16):
                wb = _bcast_lane(w16, l)
                row = g * 16 + l
                for j in range(D // 16):
                    rows[u][row, pl.ds(j * 16, 16)] = (
                        rows[u][row, pl.ds(j * 16, 16)] * wb)
            return carry
        lax.fori_loop(0, K // 16, group, 0)

    fetch_start(0, 0)
    fetch_start(1, 1)

    pltpu.sync_copy(zeros_hbm, acc_sh.at[pl.ds(s * ROWS_PER_SUB, ROWS_PER_SUB)])
    for j in range(8):
        wz_v[pl.ds(j * 16, 16)] = jnp.zeros((16,), jnp.float32)
    for t in range(ROWS_PER_SUB // 128):
        pltpu.sync_copy(wz_v, accw_sh.at[pl.ds(s * ROWS_PER_SUB + t * 128, 128)])
    plsc.subcore_barrier()

    fetch_wait(0)
    gather_start(0, 0)

    def six(t, carry):
        for k in range(6):
            i = 6 * t + k
            u = k % 3
            f = k % 2

            @pl.when(i < ch_local)
            def _():
                gather_wait(u)

            if k >= 2:
                @pl.when(i < ch_local)
                def _():
                    scatter_wait((u + 1) % 3)
            else:
                @pl.when(jnp.logical_and(t >= 1, i < ch_local))
                def _():
                    scatter_wait((u + 1) % 3)

            @pl.when(i + 1 < ch_local)
            def _():
                fetch_wait(1 - f)
                gather_start(1 - f, (u + 1) % 3)

            @pl.when(i < ch_local)
            def _():
                compute(f, u)
                scatter_start(u)

            @pl.when(i + 2 < ch_local)
            def _():
                fetch_start(i + 2, f)
        return carry
    lax.fori_loop(0, (ch_local + 5) // 6, six, 0)

    @pl.when(c == 0)
    def _():
        scatter_wait((CH_C0 - 2) % 3)
        scatter_wait((CH_C0 - 1) % 3)

    @pl.when(c == 1)
    def _():
        scatter_wait((CH_C1 - 2) % 3)
        scatter_wait((CH_C1 - 1) % 3)

    plsc.subcore_barrier()
    pltpu.sync_copy(acc_sh.at[pl.ds(s * ROWS_PER_SUB, ROWS_PER_SUB)],
                    out_hbm.at[c, pl.ds(s * ROWS_PER_SUB, ROWS_PER_SUB)])
    for t in range(ROWS_PER_SUB // 128):
        pltpu.sync_copy(accw_sh.at[pl.ds(s * ROWS_PER_SUB + t * 128, 128)],
                        wz_v)
        pltpu.sync_copy(wz_v, outw_hbm.at[pl.ds(
            c * N_ACC + s * ROWS_PER_SUB + t * 128, 128)])


_BR = 1000


def _tc_pre(x, w1, w2, b):
    return pl.pallas_call(
        _tc_pre_body,
        grid=(N // _BR,),
        in_specs=[
            pl.BlockSpec((_BR, D), lambda i: (i, 0)),
            pl.BlockSpec((D, D), lambda i: (0, 0)),
            pl.BlockSpec((D, D), lambda i: (0, 0)),
            pl.BlockSpec((1, D), lambda i: (0, 0)),
        ],
        out_specs=[
            pl.BlockSpec((_BR, D), lambda i: (i, 0)),
            pl.BlockSpec((_BR, D), lambda i: (i, 0)),
        ],
        out_shape=[
            jax.ShapeDtypeStruct((N, D), jnp.float32),
            jax.ShapeDtypeStruct((N, D), jnp.float32),
        ],
    )(x, w1, w2, b)


_sc_agg = functools.partial(
    pl.kernel,
    out_type=[
        jax.ShapeDtypeStruct((2, N_ACC, D), jnp.float32),
        jax.ShapeDtypeStruct((2 * N_ACC,), jnp.float32),
    ],
    mesh=plsc.VectorSubcoreMesh(core_axis_name="c", subcore_axis_name="s"),
    scratch_types=[
        pltpu.VMEM((K, D), jnp.float32),
        pltpu.VMEM((K, D), jnp.float32),
        pltpu.VMEM((K, D), jnp.float32),
        pltpu.VMEM((K,), jnp.int32),
        pltpu.VMEM((K,), jnp.int32),
        pltpu.VMEM((K,), jnp.int32),
        pltpu.VMEM((K,), jnp.float32),
        pltpu.VMEM((K,), jnp.float32),
        pltpu.VMEM((K,), jnp.float32),
        pltpu.VMEM((K,), jnp.int32),
        pltpu.VMEM((K,), jnp.int32),
        pltpu.VMEM((K,), jnp.int32),
        pltpu.VMEM((K,), jnp.int32),
        pltpu.VMEM((K,), jnp.float32),
        pltpu.VMEM((K,), jnp.float32),
        pltpu.VMEM((128,), jnp.float32),
        pltpu.VMEM_SHARED((N_ACC, D), jnp.float32),
        pltpu.VMEM_SHARED((N_ACC,), jnp.float32),
        pltpu.SemaphoreType.DMA,
        pltpu.SemaphoreType.DMA,
        pltpu.SemaphoreType.DMA,
        pltpu.SemaphoreType.DMA,
        pltpu.SemaphoreType.DMA,
        pltpu.SemaphoreType.DMA,
        pltpu.SemaphoreType.DMA,
        pltpu.SemaphoreType.DMA,
        pltpu.SemaphoreType.DMA,
        pltpu.SemaphoreType.DMA,
        pltpu.SemaphoreType.DMA,
    ],
)(_sc_agg_body)


def _tc_post(h1, acc, accw):
    return pl.pallas_call(
        _tc_post_body,
        out_shape=jax.ShapeDtypeStruct((N, D), jnp.float32),
    )(h1, acc, accw)


def kernel(x, edge_index, edge_weight, kernel_1, kernel_2, bias):
    ei = edge_index.astype(jnp.int32).reshape(2 * E)
    w = edge_weight.astype(jnp.float32)
    zeros = jnp.zeros((ROWS_PER_SUB, D), jnp.float32)
    h1, h2 = _tc_pre(x, kernel_1, kernel_2, bias.reshape(1, D))
    acc, accw_flat = _sc_agg(h2, ei, w, zeros)
    return _tc_post(h1, acc, accw_flat.reshape(2, N_ACC))
